# baseline (speedup 1.0000x reference)
import jax
import jax.numpy as jnp
from jax.experimental import pallas as pl
from jax.experimental.pallas import tpu as pltpu

N = 10000
D = 128
TM = 2560
TN = 2560
NPAD = 10240
NI = NPAD // TM
NJ = NPAD // TN

_REG = 2.0 / 3.0


def _norm_kernel(x_ref, f_ref, v_ref):
    x = x_ref[...]
    nrm2 = jnp.sum(x * x, axis=1, keepdims=True)
    f = x * jax.lax.rsqrt(jnp.maximum(nrm2, 1e-24))
    f_ref[...] = f.astype(jnp.bfloat16)
    v_ref[...] = jnp.broadcast_to(jnp.sum(f * f, axis=1, keepdims=True),
                                  v_ref.shape)


def _relu_sim(f_i, f_j):
    s = jax.lax.dot_general(
        f_i, f_j, (((1,), (1,)), ((), ())),
        preferred_element_type=jnp.float32)
    return jnp.maximum(s, 0.0)


def _rowsum_kernel(f_i_ref, f_j_ref, rs_ref, cs_ref, acc_ref):
    i = pl.program_id(0)
    j = pl.program_id(1)

    @pl.when(jnp.logical_and(i == 0, j == 0))
    def _():
        cs_ref[...] = jnp.zeros_like(cs_ref)

    @pl.when(j == i)
    def _():
        acc_ref[...] = jnp.zeros_like(acc_ref)

    @pl.when(j >= i)
    def _():
        s = _relu_sim(f_i_ref[...], f_j_ref[...])
        ps = s[:, 0:D]
        for k in range(1, TN // D):
            ps = ps + s[:, k * D:(k + 1) * D]
        acc_ref[...] += ps

        @pl.when(j > i)
        def _():
            cs = jnp.sum(s, axis=0, keepdims=True)
            cs_ref[pl.ds(j, 1), :] += cs

    @pl.when(j == NJ - 1)
    def _():
        rs_ref[...] = acc_ref[...]


def _combine_kernel(x_ref, rs_ref, cs_ref, v_ref, d_ref, y_ref):
    rows = jnp.sum(rs_ref[...], axis=1, keepdims=True)
    d = jax.lax.rsqrt(jnp.broadcast_to(rows, d_ref.shape) + cs_ref[...]
                      - v_ref[...] + 1.0)
    d_ref[...] = d
    y_ref[...] = (d * x_ref[...]).astype(jnp.bfloat16)


def _diffuse_kernel(x_i_ref, f_i_ref, f_j_ref, d_i_ref, y_j_ref, v_i_ref,
                    o_ref, acc_ref):
    j = pl.program_id(1)

    @pl.when(j == 0)
    def _():
        acc_ref[...] = jnp.zeros_like(acc_ref)

    s = _relu_sim(f_i_ref[...], f_j_ref[...])
    acc_ref[...] += jax.lax.dot_general(
        s.astype(jnp.bfloat16), y_j_ref[...], (((1,), (0,)), ((), ())),
        preferred_element_type=jnp.float32)

    @pl.when(j == NJ - 1)
    def _():
        x_i = x_i_ref[...]
        d_i = d_i_ref[...]
        y_i = d_i * x_i
        adj = acc_ref[...] + (1.0 - v_i_ref[...]) * y_i
        o_ref[...] = (1.0 - _REG) * x_i + _REG * d_i * adj


def _spec_i(bs):
    return pl.BlockSpec(bs, lambda i, j: (i, 0))


def _spec_j(bs):
    return pl.BlockSpec(bs, lambda i, j: (j, 0))


@jax.jit
def kernel(X):
    Xp = jnp.pad(X, ((0, NPAD - N), (0, 0)))

    F, V = pl.pallas_call(
        _norm_kernel,
        grid=(NI,),
        in_specs=[pl.BlockSpec((TM, D), lambda i: (i, 0))],
        out_specs=(pl.BlockSpec((TM, D), lambda i: (i, 0)),
                   pl.BlockSpec((TM, D), lambda i: (i, 0))),
        out_shape=(jax.ShapeDtypeStruct((NPAD, D), jnp.bfloat16),
                   jax.ShapeDtypeStruct((NPAD, D), jnp.float32)),
    )(Xp)

    RS, CS = pl.pallas_call(
        _rowsum_kernel,
        grid=(NI, NJ),
        in_specs=[_spec_i((TM, D)), _spec_j((TN, D))],
        out_specs=(_spec_i((TM, D)),
                   pl.BlockSpec((NJ, TN), lambda i, j: (0, 0))),
        out_shape=(jax.ShapeDtypeStruct((NPAD, D), jnp.float32),
                   jax.ShapeDtypeStruct((NJ, TN), jnp.float32)),
        scratch_shapes=[pltpu.VMEM((TM, D), jnp.float32)],
        compiler_params=pltpu.CompilerParams(
            dimension_semantics=("arbitrary", "arbitrary")),
    )(F, F)

    CSB = jnp.broadcast_to(jnp.reshape(CS, (NPAD,))[:, None], (NPAD, D))

    DB, YB = pl.pallas_call(
        _combine_kernel,
        grid=(NI,),
        in_specs=[pl.BlockSpec((TM, D), lambda i: (i, 0))] * 4,
        out_specs=(pl.BlockSpec((TM, D), lambda i: (i, 0)),
                   pl.BlockSpec((TM, D), lambda i: (i, 0))),
        out_shape=(jax.ShapeDtypeStruct((NPAD, D), jnp.float32),
                   jax.ShapeDtypeStruct((NPAD, D), jnp.bfloat16)),
    )(Xp, RS, CSB, V)

    OUT = pl.pallas_call(
        _diffuse_kernel,
        grid=(NI, NJ),
        in_specs=[_spec_i((TM, D)), _spec_i((TM, D)), _spec_j((TN, D)),
                  _spec_i((TM, D)), _spec_j((TN, D)), _spec_i((TM, D))],
        out_specs=_spec_i((TM, D)),
        out_shape=jax.ShapeDtypeStruct((NPAD, D), jnp.float32),
        scratch_shapes=[pltpu.VMEM((TM, D), jnp.float32)],
        compiler_params=pltpu.CompilerParams(
            dimension_semantics=("parallel", "arbitrary")),
    )(Xp, F, F, DB, YB, V)

    return OUT[:N]


# speedup vs baseline: 1.0676x; 1.0676x over previous
import jax
import jax.numpy as jnp
from jax.experimental import pallas as pl
from jax.experimental.pallas import tpu as pltpu

N = 10000
D = 128
TM = 2560
TN = 5120
NPAD = 10240
NI = NPAD // TM
NJ = NPAD // TN

_REG = 2.0 / 3.0


def _norm_kernel(x_ref, f_ref, v_ref):
    x = x_ref[...]
    nrm2 = jnp.sum(x * x, axis=1, keepdims=True)
    f = x * jax.lax.rsqrt(jnp.maximum(nrm2, 1e-24))
    f_ref[...] = f.astype(jnp.bfloat16)
    v_ref[...] = jnp.broadcast_to(jnp.sum(f * f, axis=1, keepdims=True),
                                  v_ref.shape)


def _relu_sim(f_i, f_j):
    s = jax.lax.dot_general(
        f_i, f_j, (((1,), (1,)), ((), ())),
        preferred_element_type=jnp.float32)
    return jnp.maximum(s, 0.0)


def _rowsum_kernel(x_i_ref, f_i_ref, f_j_ref, v_i_ref,
                   d_ref, y_ref, acc_ref):
    j = pl.program_id(1)

    @pl.when(j == 0)
    def _():
        acc_ref[...] = jnp.zeros_like(acc_ref)

    s = _relu_sim(f_i_ref[...], f_j_ref[...])
    ps = s[:, 0:D]
    for k in range(1, TN // D):
        ps = ps + s[:, k * D:(k + 1) * D]
    acc_ref[...] += ps

    @pl.when(j == NJ - 1)
    def _():
        rowsum = jnp.sum(acc_ref[...], axis=1, keepdims=True)
        d = jax.lax.rsqrt(jnp.broadcast_to(rowsum, d_ref.shape)
                          - v_i_ref[...] + 1.0)
        d_ref[...] = d
        y_ref[...] = (d * x_i_ref[...]).astype(jnp.bfloat16)


def _diffuse_kernel(x_i_ref, f_i_ref, f_j_ref, d_i_ref, y_j_ref, v_i_ref,
                    o_ref, acc_ref):
    j = pl.program_id(1)

    @pl.when(j == 0)
    def _():
        acc_ref[...] = jnp.zeros_like(acc_ref)

    s = _relu_sim(f_i_ref[...], f_j_ref[...])
    acc_ref[...] += jax.lax.dot_general(
        s.astype(jnp.bfloat16), y_j_ref[...], (((1,), (0,)), ((), ())),
        preferred_element_type=jnp.float32)

    @pl.when(j == NJ - 1)
    def _():
        x_i = x_i_ref[...]
        d_i = d_i_ref[...]
        y_i = d_i * x_i
        adj = acc_ref[...] + (1.0 - v_i_ref[...]) * y_i
        o_ref[...] = (1.0 - _REG) * x_i + _REG * d_i * adj


def _spec_i(bs):
    return pl.BlockSpec(bs, lambda i, j: (i, 0))


def _spec_j(bs):
    return pl.BlockSpec(bs, lambda i, j: (j, 0))


@jax.jit
def kernel(X):
    Xp = jnp.pad(X, ((0, NPAD - N), (0, 0)))

    F, V = pl.pallas_call(
        _norm_kernel,
        grid=(NI,),
        in_specs=[pl.BlockSpec((TM, D), lambda i: (i, 0))],
        out_specs=(pl.BlockSpec((TM, D), lambda i: (i, 0)),
                   pl.BlockSpec((TM, D), lambda i: (i, 0))),
        out_shape=(jax.ShapeDtypeStruct((NPAD, D), jnp.bfloat16),
                   jax.ShapeDtypeStruct((NPAD, D), jnp.float32)),
    )(Xp)

    DB, YB = pl.pallas_call(
        _rowsum_kernel,
        grid=(NI, NJ),
        in_specs=[_spec_i((TM, D)), _spec_i((TM, D)), _spec_j((TN, D)),
                  _spec_i((TM, D))],
        out_specs=(_spec_i((TM, D)), _spec_i((TM, D))),
        out_shape=(jax.ShapeDtypeStruct((NPAD, D), jnp.float32),
                   jax.ShapeDtypeStruct((NPAD, D), jnp.bfloat16)),
        scratch_shapes=[pltpu.VMEM((TM, D), jnp.float32)],
        compiler_params=pltpu.CompilerParams(
            dimension_semantics=("parallel", "arbitrary")),
    )(Xp, F, F, V)

    OUT = pl.pallas_call(
        _diffuse_kernel,
        grid=(NI, NJ),
        in_specs=[_spec_i((TM, D)), _spec_i((TM, D)), _spec_j((TN, D)),
                  _spec_i((TM, D)), _spec_j((TN, D)), _spec_i((TM, D))],
        out_specs=_spec_i((TM, D)),
        out_shape=jax.ShapeDtypeStruct((NPAD, D), jnp.float32),
        scratch_shapes=[pltpu.VMEM((TM, D), jnp.float32)],
        compiler_params=pltpu.CompilerParams(
            dimension_semantics=("parallel", "arbitrary")),
    )(Xp, F, F, DB, YB, V)

    return OUT[:N]
